# Initial kernel scaffold; baseline (speedup 1.0000x reference)
import jax
import jax.numpy as jnp
from jax.experimental import pallas as pl


def kernel(x, edge_index, edge_weight, W1, b1, W2, b2):
    raise NotImplementedError("write your pallas kernel here")


# speedup vs baseline: 11.2244x; 11.2244x over previous
import functools

import jax
import jax.numpy as jnp
from jax import lax
from jax.experimental import pallas as pl
from jax.experimental.pallas import tpu as pltpu
from jax.experimental.pallas import tpu_sc as plsc

N = 10000
D = 128
E = 320000
NP = 10240
NC = 2
NS = 16
NW = NC * NS
CH = 128
KC = 79
EPW = KC * CH
EPAD = NW * EPW
RPT = NP // NS

_mesh = plsc.VectorSubcoreMesh(core_axis_name="c", subcore_axis_name="s")
_sc_params = pltpu.CompilerParams(needs_layout_passes=False)


@functools.partial(
    pl.kernel,
    out_type=jax.ShapeDtypeStruct((NW, NP), jnp.float32),
    mesh=_mesh,
    scratch_types=[
        pltpu.VMEM((EPW,), jnp.int32),
        pltpu.VMEM((EPW,), jnp.float32),
        pltpu.VMEM((NP,), jnp.float32),
    ],
    compiler_params=_sc_params,
)
def _deg_kernel(colf_hbm, ewf_hbm, degp_hbm, colv, ewv, acc):
    wid = lax.axis_index("s") * NC + lax.axis_index("c")
    pltpu.sync_copy(colf_hbm.at[wid], colv)
    pltpu.sync_copy(ewf_hbm.at[wid], ewv)

    def zero_body(i, _):
        acc[pl.ds(pl.multiple_of(i * 16, 16), 16)] = jnp.zeros((16,), jnp.float32)
        return 0
    lax.fori_loop(0, NP // 16, zero_body, 0)

    def body(i, _):
        sl = pl.ds(pl.multiple_of(i * 16, 16), 16)
        plsc.addupdate_scatter(acc, [colv[sl]], ewv[sl])
        return 0
    lax.fori_loop(0, EPW // 16, body, 0)
    pltpu.sync_copy(acc, degp_hbm.at[wid])


@functools.partial(
    pl.kernel,
    out_type=jax.ShapeDtypeStruct((NC, NP, D), jnp.float32),
    mesh=_mesh,
    scratch_types=[
        pltpu.VMEM((KC, CH), jnp.int32),
        pltpu.VMEM((KC, CH), jnp.int32),
        pltpu.VMEM((EPW,), jnp.float32),
        pltpu.VMEM((CH, D), jnp.float32),
        pltpu.VMEM_SHARED((NP, D), jnp.float32),
    ],
    compiler_params=_sc_params,
)
def _gs_kernel(hp_hbm, row3_hbm, col3_hbm, ewf_hbm, out_hbm,
               rowv, colv, ewv, rows, acc):
    cid = lax.axis_index("c")
    sid = lax.axis_index("s")
    wid = sid * NC + cid
    pltpu.sync_copy(row3_hbm.at[wid], rowv)
    pltpu.sync_copy(col3_hbm.at[wid], colv)
    pltpu.sync_copy(ewf_hbm.at[wid], ewv)

    def zrow(i, _):
        r = i // 8
        j = (i % 8) * 16
        rows[r, pl.ds(pl.multiple_of(j, 16), 16)] = jnp.zeros((16,), jnp.float32)
        return 0
    lax.fori_loop(0, CH * 8, zrow, 0)
    for t in range(RPT // CH):
        pltpu.sync_copy(rows, acc.at[pl.ds(sid * RPT + t * CH, CH)])
    plsc.subcore_barrier()

    def chunk(c, _):
        pltpu.sync_copy(hp_hbm.at[rowv.at[c]], rows)

        def edge(e, _):
            sv = plsc.load_gather(ewv, [jnp.full((16,), c * CH + e, jnp.int32)])
            for j in range(8):
                sl = pl.ds(j * 16, 16)
                rows[e, sl] = rows[e, sl] * sv
            return 0
        lax.fori_loop(0, CH, edge, 0)

        pltpu.sync_copy(rows, acc.at[colv.at[c]], add=True)
        return 0
    lax.fori_loop(0, KC, chunk, 0)

    plsc.subcore_barrier()
    sl = pl.ds(sid * RPT, RPT)
    pltpu.sync_copy(acc.at[sl], out_hbm.at[cid, sl])


def _mkdis_body(degp_ref, dis_ref):
    s = jnp.sum(degp_ref[...], axis=0, keepdims=True) + 1.0
    dis_ref[...] = lax.rsqrt(s)


_mkdis = pl.pallas_call(
    _mkdis_body,
    out_shape=jax.ShapeDtypeStruct((1, NP), jnp.float32),
)


def _scalemm_body(x_ref, w_ref, dis_ref, o_ref):
    h = jnp.dot(x_ref[...], w_ref[...], preferred_element_type=jnp.float32)
    o_ref[...] = dis_ref[...] * h


_scalemm = pl.pallas_call(
    _scalemm_body,
    out_shape=jax.ShapeDtypeStruct((NP, D), jnp.float32),
)


def _combine_body(s_ref, hp_ref, dis_ref, b_ref, w_ref, o_ref):
    t = s_ref[0] + s_ref[1] + hp_ref[...]
    z = dis_ref[...] * t + b_ref[...]
    h2 = jnp.maximum(z, 0.0)
    h = jnp.dot(h2, w_ref[...], preferred_element_type=jnp.float32)
    o_ref[...] = dis_ref[...] * h


_combine = pl.pallas_call(
    _combine_body,
    out_shape=jax.ShapeDtypeStruct((NP, D), jnp.float32),
)


def _final_body(s_ref, hp_ref, dis_ref, b_ref, o_ref):
    t = s_ref[0] + s_ref[1] + hp_ref[...]
    o_ref[...] = dis_ref[...] * t + b_ref[...]


_final = pl.pallas_call(
    _final_body,
    out_shape=jax.ShapeDtypeStruct((NP, D), jnp.float32),
)


def kernel(x, edge_index, edge_weight, W1, b1, W2, b2):
    row = edge_index[0].astype(jnp.int32)
    col = edge_index[1].astype(jnp.int32)
    ew = edge_weight.astype(jnp.float32)
    pad = EPAD - E
    rowp = jnp.concatenate([row, jnp.zeros((pad,), jnp.int32)])
    colp = jnp.concatenate([col, jnp.zeros((pad,), jnp.int32)])
    ewp = jnp.concatenate([ew, jnp.zeros((pad,), jnp.float32)])
    row3 = rowp.reshape(NW, KC, CH)
    col3 = colp.reshape(NW, KC, CH)
    colf = colp.reshape(NW, EPW)
    ewf = ewp.reshape(NW, EPW)
    xp = jnp.pad(x, ((0, NP - N), (0, 0)))
    b1r = b1.reshape(1, D)
    b2r = b2.reshape(1, D)

    degp = _deg_kernel(colf, ewf)
    dis = _mkdis(degp)
    dis2 = dis.reshape(NP, 1)
    h1p = _scalemm(xp, W1, dis2)
    s1 = _gs_kernel(h1p, row3, col3, ewf)
    h2p = _combine(s1, h1p, dis2, b1r, W2)
    s2 = _gs_kernel(h2p, row3, col3, ewf)
    out = _final(s2, h2p, dis2, b2r)
    return out[:N]
